# baseline (speedup 1.0000x reference)
import functools

import jax
import jax.numpy as jnp
from jax.experimental import pallas as pl
from jax.experimental.pallas import tpu as pltpu

_NUM_CENTROIDS = 1024
_EMBED_DIM = 64
_COMMITMENT_LOSS = 0.25
_EMA_DECAY = 0.99

_ROWS = 9216
_BLOCK = 2304
_SUB = 576
_NSUB = _BLOCK // _SUB
_LANES = 128
_NCHUNK = _NUM_CENTROIDS // _LANES


def _vq_kernel(train_ref, x_ref, cb_ref, cc_ref,
               q_ref, loss_ref, idx_ref, counts_ref):
    i = pl.program_id(0)
    nsteps = pl.num_programs(0)
    cb = cb_ref[...]
    cb16 = cb.astype(jnp.bfloat16)
    sc = jnp.sum(cb * cb, axis=1)[None, :]

    def subtile(s, counts):
        r0 = s * _SUB
        x = x_ref[pl.ds(r0, _SUB), :]
        sx = jnp.sum(x * x, axis=1, keepdims=True)
        xm2 = (x * -2.0).astype(jnp.bfloat16)

        best_v = jnp.full((_SUB, 1), jnp.inf, jnp.float32)
        best_i = jnp.zeros((_SUB, 1), jnp.int32)
        for j in range(_NCHUNK):
            mm2 = jax.lax.dot_general(
                xm2, cb16[j * _LANES:(j + 1) * _LANES, :],
                (((1,), (1,)), ((), ())),
                preferred_element_type=jnp.float32)
            d = sx + mm2 + sc[:, j * _LANES:(j + 1) * _LANES]
            v = jnp.min(d, axis=1, keepdims=True)
            a = jnp.argmin(d, axis=1).astype(jnp.int32)[:, None] + j * _LANES
            upd = v < best_v
            best_v = jnp.where(upd, v, best_v)
            best_i = jnp.where(upd, a, best_i)

        idx = best_i[:, 0]
        idx_ref[0, pl.ds(s, 1), :] = idx[None, :]

        iota = jax.lax.broadcasted_iota(jnp.int32, (_SUB, _NUM_CENTROIDS), 1)
        onehot = (iota == best_i).astype(jnp.float32)
        q = jax.lax.dot_general(
            onehot, cb, (((1,), (0,)), ((), ())),
            precision=jax.lax.Precision.DEFAULT,
            preferred_element_type=jnp.float32)

        dqx = q - x
        q_ref[pl.ds(r0, _SUB), :] = x + dqx
        loss_ref[pl.ds(r0, _SUB), :] = (1.0 + _COMMITMENT_LOSS) * (dqx * dqx)
        return counts + jnp.sum(onehot, axis=0)[None, :]

    part = jax.lax.fori_loop(
        0, _NSUB, subtile, jnp.zeros((1, _NUM_CENTROIDS), jnp.float32))

    @pl.when(i == 0)
    def _init():
        counts_ref[...] = jnp.zeros_like(counts_ref)

    counts_ref[...] += part

    @pl.when(i == nsteps - 1)
    def _finalize():
        t = train_ref[0]
        cc = cc_ref[...]
        cnt = counts_ref[...]
        ema = _EMA_DECAY * cc + (1.0 - _EMA_DECAY) * cnt
        counts_ref[...] = jnp.where(t != 0, ema, cc)


@functools.partial(jax.jit, static_argnames=("interpret",))
def _vq(flat_x, train_f32, codebook, cluster_counts, interpret=False):
    nblocks = _ROWS // _BLOCK
    out_shapes = (
        jax.ShapeDtypeStruct((_ROWS, _EMBED_DIM), jnp.float32),
        jax.ShapeDtypeStruct((_ROWS, _EMBED_DIM), jnp.float32),
        jax.ShapeDtypeStruct((_ROWS // _BLOCK, _NSUB, _SUB), jnp.int32),
        jax.ShapeDtypeStruct((1, _NUM_CENTROIDS), jnp.float32),
    )
    in_specs = [
        pl.BlockSpec((1,), lambda i: (0,)),
        pl.BlockSpec((_BLOCK, _EMBED_DIM), lambda i: (i, 0)),
        pl.BlockSpec((_NUM_CENTROIDS, _EMBED_DIM), lambda i: (0, 0)),
        pl.BlockSpec((1, _NUM_CENTROIDS), lambda i: (0, 0)),
    ]
    out_specs = (
        pl.BlockSpec((_BLOCK, _EMBED_DIM), lambda i: (i, 0)),
        pl.BlockSpec((_BLOCK, _EMBED_DIM), lambda i: (i, 0)),
        pl.BlockSpec((1, _NSUB, _SUB), lambda i: (i, 0, 0)),
        pl.BlockSpec((1, _NUM_CENTROIDS), lambda i: (0, 0)),
    )
    return pl.pallas_call(
        _vq_kernel,
        grid=(nblocks,),
        in_specs=in_specs,
        out_specs=out_specs,
        out_shape=out_shapes,
        compiler_params=pltpu.CompilerParams(
            dimension_semantics=("arbitrary",)),
        interpret=interpret,
    )(train_f32, flat_x, codebook, cluster_counts.reshape(1, -1))


def kernel(inputs, train, codebook, cluster_counts):
    embedding_dim = inputs.shape[-1]
    flat_x = jnp.reshape(inputs, (-1, embedding_dim))
    train_f32 = jnp.asarray(train, jnp.float32).reshape(1)
    q, loss, idx, counts = _vq(flat_x, train_f32, codebook, cluster_counts)
    quantized = jnp.reshape(q, inputs.shape)
    quantization_loss = jnp.reshape(loss, inputs.shape)
    nn_idx = jnp.reshape(idx, (1,) + inputs.shape[:-1])
    codebook_values = jax.lax.stop_gradient(codebook[None])
    new_counts = counts.reshape(-1)
    return (quantized, quantization_loss, nn_idx, codebook_values, new_counts)


# speedup vs baseline: 1.1298x; 1.1298x over previous
import functools

import jax
import jax.numpy as jnp
from jax.experimental import pallas as pl
from jax.experimental.pallas import tpu as pltpu

_NUM_CENTROIDS = 1024
_EMBED_DIM = 64
_COMMITMENT_LOSS = 0.25
_EMA_DECAY = 0.99


def _vq_kernel(train_ref, x_ref, cb_ref, cc_ref,
               q_ref, loss_ref, idx_ref, counts_ref, sc_ref):
    i = pl.program_id(0)
    nsteps = pl.num_programs(0)
    x = x_ref[...]
    cb = cb_ref[...]

    @pl.when(i == 0)
    def _sc_init():
        sc_ref[...] = jnp.sum(cb * cb, axis=1)[None, :]

    sx = jnp.sum(x * x, axis=1, keepdims=True)
    sc = sc_ref[...]
    mm2 = jax.lax.dot_general(
        (x * -2.0).astype(jnp.bfloat16), cb.astype(jnp.bfloat16),
        (((1,), (1,)), ((), ())),
        preferred_element_type=jnp.float32)
    d = sx + mm2 + sc

    idx = jnp.argmin(d, axis=1).astype(jnp.int32)
    idx_ref[0, 0, :] = idx

    iota = jax.lax.broadcasted_iota(jnp.int32, d.shape, 1)
    onehot = (iota == idx[:, None]).astype(jnp.float32)
    q = jax.lax.dot_general(
        onehot, cb, (((1,), (0,)), ((), ())),
        precision=jax.lax.Precision.DEFAULT,
        preferred_element_type=jnp.float32)

    dqx = q - x
    q_ref[...] = x + dqx
    loss_ref[...] = (1.0 + _COMMITMENT_LOSS) * (dqx * dqx)

    part = jnp.sum(onehot, axis=0)[None, :]

    @pl.when(i == 0)
    def _init():
        counts_ref[...] = jnp.zeros_like(counts_ref)

    counts_ref[...] += part

    @pl.when(i == nsteps - 1)
    def _finalize():
        t = train_ref[0]
        cc = cc_ref[...]
        cnt = counts_ref[...]
        ema = _EMA_DECAY * cc + (1.0 - _EMA_DECAY) * cnt
        counts_ref[...] = jnp.where(t != 0, ema, cc)


@functools.partial(jax.jit, static_argnames=("block_rows", "interpret"))
def _vq(flat_x, train_f32, codebook, cluster_counts,
        block_rows=2304, interpret=False):
    rows = flat_x.shape[0]
    nblocks = rows // block_rows
    grid = (nblocks,)
    out_shapes = (
        jax.ShapeDtypeStruct((rows, _EMBED_DIM), jnp.float32),
        jax.ShapeDtypeStruct((rows, _EMBED_DIM), jnp.float32),
        jax.ShapeDtypeStruct((nblocks, 1, block_rows), jnp.int32),
        jax.ShapeDtypeStruct((1, _NUM_CENTROIDS), jnp.float32),
    )
    in_specs = [
        pl.BlockSpec((1,), lambda i: (0,)),
        pl.BlockSpec((block_rows, _EMBED_DIM), lambda i: (i, 0)),
        pl.BlockSpec((_NUM_CENTROIDS, _EMBED_DIM), lambda i: (0, 0)),
        pl.BlockSpec((1, _NUM_CENTROIDS), lambda i: (0, 0)),
    ]
    out_specs = (
        pl.BlockSpec((block_rows, _EMBED_DIM), lambda i: (i, 0)),
        pl.BlockSpec((block_rows, _EMBED_DIM), lambda i: (i, 0)),
        pl.BlockSpec((1, 1, block_rows), lambda i: (i, 0, 0)),
        pl.BlockSpec((1, _NUM_CENTROIDS), lambda i: (0, 0)),
    )
    return pl.pallas_call(
        _vq_kernel,
        grid=grid,
        in_specs=in_specs,
        out_specs=out_specs,
        out_shape=out_shapes,
        scratch_shapes=[pltpu.VMEM((1, _NUM_CENTROIDS), jnp.float32)],
        compiler_params=pltpu.CompilerParams(
            dimension_semantics=("arbitrary",)),
        interpret=interpret,
    )(train_f32, flat_x, codebook, cluster_counts.reshape(1, -1))


def kernel(inputs, train, codebook, cluster_counts):
    embedding_dim = inputs.shape[-1]
    flat_x = jnp.reshape(inputs, (-1, embedding_dim))
    train_f32 = jnp.asarray(train, jnp.float32).reshape(1)
    q, loss, idx, counts = _vq(flat_x, train_f32, codebook, cluster_counts)
    quantized = jnp.reshape(q, inputs.shape)
    quantization_loss = jnp.reshape(loss, inputs.shape)
    nn_idx = jnp.reshape(idx, (1,) + inputs.shape[:-1])
    codebook_values = jax.lax.stop_gradient(codebook[None])
    new_counts = counts.reshape(-1)
    return (quantized, quantization_loss, nn_idx, codebook_values, new_counts)
